# Initial kernel scaffold; baseline (speedup 1.0000x reference)
import jax
import jax.numpy as jnp
from jax.experimental import pallas as pl


def kernel(node_obs, adj, W1, b1, W2, b2, W3, b3, W4, b4, W5, b5):
    raise NotImplementedError("write your pallas kernel here")


# speedup vs baseline: 9.2644x; 9.2644x over previous
import functools

import jax
import jax.numpy as jnp
from jax.experimental import pallas as pl
from jax.experimental.pallas import tpu as pltpu

B, N, IN_F, OUT_F = 16, 1024, 64, 64
SENS = 0.05


def _gnn_batch_kernel(x_ref, a_ref, w1_ref, b1_ref, w2_ref, b2_ref,
                      w3_ref, b3_ref, w4_ref, b4_ref, w5_ref, b5_ref,
                      out_ref):
    x = x_ref[0]
    a = a_ref[0]
    inv_scale = jnp.float32(1.0) / jnp.sqrt(jnp.float32(OUT_F))

    t2 = jax.lax.dot_general(x, w2_ref[...], (((1,), (1,)), ((), ())),
                             preferred_element_type=jnp.float32,
                             precision=jax.lax.Precision.HIGHEST) + b2_ref[...]
    t3 = jax.lax.dot_general(x, w3_ref[...], (((1,), (1,)), ((), ())),
                             preferred_element_type=jnp.float32,
                             precision=jax.lax.Precision.HIGHEST) + b3_ref[...]
    t4 = jax.lax.dot_general(x, w4_ref[...], (((1,), (1,)), ((), ())),
                             preferred_element_type=jnp.float32,
                             precision=jax.lax.Precision.HIGHEST) + b4_ref[...]

    u = jnp.sum(t3 * w5_ref[...], axis=1, keepdims=True) * inv_scale

    d = jax.lax.dot_general(t4, t3, (((1,), (1,)), ((), ())),
                            preferred_element_type=jnp.float32,
                            precision=jax.lax.Precision.HIGHEST) * inv_scale

    mask = (a < SENS) & (a > 0)
    s = d + a * u.T
    neg_inf = jnp.float32(-jnp.inf)
    sm = jnp.where(mask, s, neg_inf)
    smax = jnp.max(sm, axis=0, keepdims=True)
    e = jnp.where(mask, jnp.exp(s - smax), jnp.float32(0.0))
    denom = jnp.sum(e, axis=0, keepdims=True)
    alpha = e * (jnp.float32(1.0) / (denom + jnp.float32(1e-16)))
    r = jnp.sum(alpha, axis=1, keepdims=True)

    contrib = jax.lax.dot_general(r, t2, (((0,), (0,)), ((), ())),
                                  preferred_element_type=jnp.float32,
                                  precision=jax.lax.Precision.HIGHEST)

    mean_x = jnp.mean(x, axis=0, keepdims=True)
    lin = jax.lax.dot_general(mean_x, w1_ref[...], (((1,), (1,)), ((), ())),
                              preferred_element_type=jnp.float32,
                              precision=jax.lax.Precision.HIGHEST) + b1_ref[...]
    out_ref[0, 0] = lin[0] + contrib[0] * (jnp.float32(1.0) / jnp.float32(N))


def kernel(node_obs, adj, W1, b1, W2, b2, W3, b3, W4, b4, W5, b5):
    w5_row = W5[:, 0][None, :]
    b5_row = b5[None, :]

    grid_spec = pl.GridSpec(
        grid=(B,),
        in_specs=[
            pl.BlockSpec((1, N, IN_F), lambda b: (b, 0, 0)),
            pl.BlockSpec((1, N, N), lambda b: (b, 0, 0)),
            pl.BlockSpec((OUT_F, IN_F), lambda b: (0, 0)),
            pl.BlockSpec((1, OUT_F), lambda b: (0, 0)),
            pl.BlockSpec((OUT_F, IN_F), lambda b: (0, 0)),
            pl.BlockSpec((1, OUT_F), lambda b: (0, 0)),
            pl.BlockSpec((OUT_F, IN_F), lambda b: (0, 0)),
            pl.BlockSpec((1, OUT_F), lambda b: (0, 0)),
            pl.BlockSpec((OUT_F, IN_F), lambda b: (0, 0)),
            pl.BlockSpec((1, OUT_F), lambda b: (0, 0)),
            pl.BlockSpec((1, OUT_F), lambda b: (0, 0)),
            pl.BlockSpec((1, OUT_F), lambda b: (0, 0)),
        ],
        out_specs=pl.BlockSpec((1, 1, OUT_F), lambda b: (b, 0, 0)),
    )

    out = pl.pallas_call(
        _gnn_batch_kernel,
        grid_spec=grid_spec,
        out_shape=jax.ShapeDtypeStruct((B, 1, OUT_F), jnp.float32),
    )(node_obs, adj, W1, b1[None, :], W2, b2[None, :], W3, b3[None, :],
      W4, b4[None, :], w5_row, b5_row)
    return out.reshape(B, OUT_F)


# speedup vs baseline: 10.2168x; 1.1028x over previous
import functools

import jax
import jax.numpy as jnp
from jax.experimental import pallas as pl
from jax.experimental.pallas import tpu as pltpu

B, N, IN_F, OUT_F = 16, 1024, 64, 64
SENS = 0.05


def _gnn_batch_kernel(x_ref, a_ref, w1_ref, b1_ref, w2_ref, b2_ref,
                      w3_ref, b3_ref, w4_ref, b4_ref, w5_ref, b5_ref,
                      out_ref):
    x = x_ref[0]
    a = a_ref[0]
    inv_scale = jnp.float32(1.0) / jnp.sqrt(jnp.float32(OUT_F))

    t2 = jax.lax.dot_general(x, w2_ref[...], (((1,), (1,)), ((), ())),
                             preferred_element_type=jnp.float32,
                             precision=jax.lax.Precision.HIGHEST) + b2_ref[...]
    t3 = jax.lax.dot_general(x, w3_ref[...], (((1,), (1,)), ((), ())),
                             preferred_element_type=jnp.float32,
                             precision=jax.lax.Precision.HIGHEST) + b3_ref[...]
    t4 = jax.lax.dot_general(x, w4_ref[...], (((1,), (1,)), ((), ())),
                             preferred_element_type=jnp.float32,
                             precision=jax.lax.Precision.HIGHEST) + b4_ref[...]

    u = jnp.sum(t3 * w5_ref[...], axis=1, keepdims=True) * inv_scale

    d = jax.lax.dot_general(t4, t3, (((1,), (1,)), ((), ())),
                            preferred_element_type=jnp.float32,
                            precision=jax.lax.Precision.HIGHEST) * inv_scale

    mask = (a < SENS) & (a > 0)
    neg_inf = jnp.float32(-jnp.inf)
    sm = jnp.where(mask, d + a * u.T, neg_inf)
    smax = jnp.max(sm, axis=0, keepdims=True)
    m = jnp.maximum(smax, jnp.float32(0.0))
    e = jnp.exp(sm - m)
    denom = jnp.sum(e, axis=0, keepdims=True)
    alpha = e * (jnp.float32(1.0) / (denom + jnp.float32(1e-16)))
    r = jnp.sum(alpha, axis=1, keepdims=True)

    contrib = jax.lax.dot_general(r, t2, (((0,), (0,)), ((), ())),
                                  preferred_element_type=jnp.float32,
                                  precision=jax.lax.Precision.HIGHEST)

    mean_x = jnp.mean(x, axis=0, keepdims=True)
    lin = jax.lax.dot_general(mean_x, w1_ref[...], (((1,), (1,)), ((), ())),
                              preferred_element_type=jnp.float32,
                              precision=jax.lax.Precision.HIGHEST) + b1_ref[...]
    out_ref[0, 0] = lin[0] + contrib[0] * (jnp.float32(1.0) / jnp.float32(N))


def kernel(node_obs, adj, W1, b1, W2, b2, W3, b3, W4, b4, W5, b5):
    w5_row = W5[:, 0][None, :]
    b5_row = b5[None, :]

    grid_spec = pl.GridSpec(
        grid=(B,),
        in_specs=[
            pl.BlockSpec((1, N, IN_F), lambda b: (b, 0, 0)),
            pl.BlockSpec((1, N, N), lambda b: (b, 0, 0)),
            pl.BlockSpec((OUT_F, IN_F), lambda b: (0, 0)),
            pl.BlockSpec((1, OUT_F), lambda b: (0, 0)),
            pl.BlockSpec((OUT_F, IN_F), lambda b: (0, 0)),
            pl.BlockSpec((1, OUT_F), lambda b: (0, 0)),
            pl.BlockSpec((OUT_F, IN_F), lambda b: (0, 0)),
            pl.BlockSpec((1, OUT_F), lambda b: (0, 0)),
            pl.BlockSpec((OUT_F, IN_F), lambda b: (0, 0)),
            pl.BlockSpec((1, OUT_F), lambda b: (0, 0)),
            pl.BlockSpec((1, OUT_F), lambda b: (0, 0)),
            pl.BlockSpec((1, OUT_F), lambda b: (0, 0)),
        ],
        out_specs=pl.BlockSpec((1, 1, OUT_F), lambda b: (b, 0, 0)),
    )

    out = pl.pallas_call(
        _gnn_batch_kernel,
        grid_spec=grid_spec,
        out_shape=jax.ShapeDtypeStruct((B, 1, OUT_F), jnp.float32),
    )(node_obs, adj, W1, b1[None, :], W2, b2[None, :], W3, b3[None, :],
      W4, b4[None, :], w5_row, b5_row)
    return out.reshape(B, OUT_F)


# speedup vs baseline: 13.4594x; 1.3174x over previous
import functools

import jax
import jax.numpy as jnp
from jax.experimental import pallas as pl
from jax.experimental.pallas import tpu as pltpu

B, N, IN_F, OUT_F = 16, 1024, 64, 64
SENS = 0.05


def _gnn_batch_kernel(x_ref, a_ref, w1_ref, b1_ref, w2_ref, b2_ref,
                      w3_ref, b3_ref, w4_ref, b4_ref, w5_ref, b5_ref,
                      out_ref):
    x = x_ref[0]
    a = a_ref[0]
    inv_scale = jnp.float32(1.0) / jnp.sqrt(jnp.float32(OUT_F))

    t2 = jax.lax.dot_general(x, w2_ref[...], (((1,), (1,)), ((), ())),
                             preferred_element_type=jnp.float32,
                             precision=jax.lax.Precision.HIGHEST) + b2_ref[...]
    t3 = jax.lax.dot_general(x, w3_ref[...], (((1,), (1,)), ((), ())),
                             preferred_element_type=jnp.float32,
                             precision=jax.lax.Precision.HIGHEST) + b3_ref[...]
    t4 = jax.lax.dot_general(x, w4_ref[...], (((1,), (1,)), ((), ())),
                             preferred_element_type=jnp.float32,
                             precision=jax.lax.Precision.HIGHEST) + b4_ref[...]

    u = jnp.sum(t3 * w5_ref[...], axis=1, keepdims=True) * inv_scale

    d = jax.lax.dot_general(t4, t3, (((1,), (1,)), ((), ())),
                            preferred_element_type=jnp.float32,
                            precision=jax.lax.Precision.DEFAULT) * inv_scale

    mask = (a < SENS) & (a > 0)
    neg_inf = jnp.float32(-jnp.inf)
    sm = jnp.where(mask, d + a * u.T, neg_inf)
    smax = jnp.max(sm, axis=0, keepdims=True)
    m = jnp.maximum(smax, jnp.float32(0.0))
    e = jnp.exp(sm - m)
    denom = jnp.sum(e, axis=0, keepdims=True)
    invd = jnp.float32(1.0) / (denom + jnp.float32(1e-16))

    g = jax.lax.dot_general(e, t2, (((0,), (0,)), ((), ())),
                            preferred_element_type=jnp.float32,
                            precision=jax.lax.Precision.DEFAULT)
    contrib = jax.lax.dot_general(invd, g, (((1,), (0,)), ((), ())),
                                  preferred_element_type=jnp.float32,
                                  precision=jax.lax.Precision.HIGHEST)

    mean_x = jnp.mean(x, axis=0, keepdims=True)
    lin = jax.lax.dot_general(mean_x, w1_ref[...], (((1,), (1,)), ((), ())),
                              preferred_element_type=jnp.float32,
                              precision=jax.lax.Precision.HIGHEST) + b1_ref[...]
    out_ref[0, 0] = lin[0] + contrib[0] * (jnp.float32(1.0) / jnp.float32(N))


def kernel(node_obs, adj, W1, b1, W2, b2, W3, b3, W4, b4, W5, b5):
    w5_row = W5[:, 0][None, :]
    b5_row = b5[None, :]

    grid_spec = pl.GridSpec(
        grid=(B,),
        in_specs=[
            pl.BlockSpec((1, N, IN_F), lambda b: (b, 0, 0)),
            pl.BlockSpec((1, N, N), lambda b: (b, 0, 0)),
            pl.BlockSpec((OUT_F, IN_F), lambda b: (0, 0)),
            pl.BlockSpec((1, OUT_F), lambda b: (0, 0)),
            pl.BlockSpec((OUT_F, IN_F), lambda b: (0, 0)),
            pl.BlockSpec((1, OUT_F), lambda b: (0, 0)),
            pl.BlockSpec((OUT_F, IN_F), lambda b: (0, 0)),
            pl.BlockSpec((1, OUT_F), lambda b: (0, 0)),
            pl.BlockSpec((OUT_F, IN_F), lambda b: (0, 0)),
            pl.BlockSpec((1, OUT_F), lambda b: (0, 0)),
            pl.BlockSpec((1, OUT_F), lambda b: (0, 0)),
            pl.BlockSpec((1, OUT_F), lambda b: (0, 0)),
        ],
        out_specs=pl.BlockSpec((1, 1, OUT_F), lambda b: (b, 0, 0)),
    )

    out = pl.pallas_call(
        _gnn_batch_kernel,
        grid_spec=grid_spec,
        out_shape=jax.ShapeDtypeStruct((B, 1, OUT_F), jnp.float32),
    )(node_obs, adj, W1, b1[None, :], W2, b2[None, :], W3, b3[None, :],
      W4, b4[None, :], w5_row, b5_row)
    return out.reshape(B, OUT_F)


# speedup vs baseline: 15.4241x; 1.1460x over previous
import functools

import jax
import jax.numpy as jnp
from jax.experimental import pallas as pl
from jax.experimental.pallas import tpu as pltpu

B, N, IN_F, OUT_F = 16, 1024, 64, 64
SENS = 0.05


def _gnn_batch_kernel(x_ref, a_ref, wp_ref, bp_ref, w1_ref, b1_ref, out_ref):
    x = x_ref[0]
    a = a_ref[0]
    inv_scale = jnp.float32(1.0) / jnp.sqrt(jnp.float32(OUT_F))

    p = jax.lax.dot_general(x, wp_ref[...], (((1,), (0,)), ((), ())),
                            preferred_element_type=jnp.float32,
                            precision=jax.lax.Precision.DEFAULT) + bp_ref[...]
    t2 = p[:, 0:OUT_F]
    t3 = p[:, OUT_F:2 * OUT_F]
    t4 = p[:, 2 * OUT_F:3 * OUT_F]
    u = p[:, 3 * OUT_F:3 * OUT_F + 1]

    d = jax.lax.dot_general(t4, t3, (((1,), (1,)), ((), ())),
                            preferred_element_type=jnp.float32,
                            precision=jax.lax.Precision.DEFAULT) * inv_scale

    mask = (a < SENS) & (a > 0)
    neg_inf = jnp.float32(-jnp.inf)
    sm = jnp.where(mask, d + a * u.T, neg_inf)
    smax = jnp.max(sm, axis=0, keepdims=True)
    m = jnp.maximum(smax, jnp.float32(0.0))
    e = jnp.exp(sm - m)
    denom = jnp.sum(e, axis=0, keepdims=True)
    invd = jnp.float32(1.0) / (denom + jnp.float32(1e-16))

    g = jax.lax.dot_general(e, t2, (((0,), (0,)), ((), ())),
                            preferred_element_type=jnp.float32,
                            precision=jax.lax.Precision.DEFAULT)
    contrib = jax.lax.dot_general(invd, g, (((1,), (0,)), ((), ())),
                                  preferred_element_type=jnp.float32,
                                  precision=jax.lax.Precision.HIGHEST)

    mean_x = jnp.mean(x, axis=0, keepdims=True)
    lin = jax.lax.dot_general(mean_x, w1_ref[...], (((1,), (1,)), ((), ())),
                              preferred_element_type=jnp.float32,
                              precision=jax.lax.Precision.HIGHEST) + b1_ref[...]
    out_ref[0, 0] = lin[0] + contrib[0] * (jnp.float32(1.0) / jnp.float32(N))


def kernel(node_obs, adj, W1, b1, W2, b2, W3, b3, W4, b4, W5, b5):
    inv_scale = 1.0 / jnp.sqrt(jnp.float32(OUT_F))
    w5c = W5[:, 0]
    v = (W3.T @ w5c) * inv_scale
    c = jnp.dot(b3, w5c) * inv_scale
    vp = jnp.zeros((IN_F, OUT_F), jnp.float32).at[:, 0].set(v)
    wp = jnp.concatenate([W2.T, W3.T, W4.T, vp], axis=1)
    bp = jnp.concatenate(
        [b2, b3, b4, jnp.zeros((OUT_F,), jnp.float32).at[0].set(c)])[None, :]

    grid_spec = pl.GridSpec(
        grid=(B,),
        in_specs=[
            pl.BlockSpec((1, N, IN_F), lambda b: (b, 0, 0)),
            pl.BlockSpec((1, N, N), lambda b: (b, 0, 0)),
            pl.BlockSpec((IN_F, 4 * OUT_F), lambda b: (0, 0)),
            pl.BlockSpec((1, 4 * OUT_F), lambda b: (0, 0)),
            pl.BlockSpec((OUT_F, IN_F), lambda b: (0, 0)),
            pl.BlockSpec((1, OUT_F), lambda b: (0, 0)),
        ],
        out_specs=pl.BlockSpec((1, 1, OUT_F), lambda b: (b, 0, 0)),
    )

    out = pl.pallas_call(
        _gnn_batch_kernel,
        grid_spec=grid_spec,
        out_shape=jax.ShapeDtypeStruct((B, 1, OUT_F), jnp.float32),
    )(node_obs, adj, wp, bp, W1, b1[None, :])
    return out.reshape(B, OUT_F)
